# baseline (speedup 1.0000x reference)
import functools

import jax
import jax.numpy as jnp
from jax import lax
from jax.experimental import pallas as pl
from jax.experimental.pallas import tpu as pltpu
from jax.experimental.pallas import tpu_sc as plsc

_BASE_ALPHA = 0.05
_CHUNK = 512


def _dist_chunk(src_ref, c, part, k2, iota_m, m):
    q = src_ref[0, pl.ds(c * _CHUNK, _CHUNK), :]
    qk = lax.dot_general(-2.0 * q, part, (((1,), (1,)), ((), ())),
                         preferred_element_type=jnp.float32)
    s = qk + k2
    mn = jnp.min(s, axis=1, keepdims=True)
    idx = jnp.min(jnp.where(s <= mn, iota_m, float(m)), axis=1,
                  keepdims=True)
    q2 = jnp.sum(q * q, axis=1, keepdims=True)
    return mn + q2, idx


def _k1_body(pred_ref, partial_ref, idx_ref, mind2_ref):
    m = partial_ref.shape[1]
    n_chunks = pred_ref.shape[1] // _CHUNK
    b = pl.program_id(0)
    part = partial_ref[0]
    k2 = jnp.sum(part * part, axis=1)[None, :]
    iota_m = lax.broadcasted_iota(jnp.int32, (_CHUNK, m), 1).astype(jnp.float32)
    for c in range(n_chunks):
        mind2, idx = _dist_chunk(pred_ref, c, part, k2, iota_m, m)
        idx_ref[0, 0, pl.ds(c * _CHUNK, _CHUNK)] = (
            idx.astype(jnp.int32) + b * m).reshape(_CHUNK)
        mind2_ref[0, :, c:c + 1] = mind2


def _update_chunk(src_ref, near_ref, mind2_ref, out_ref, c, denom):
    q = src_ref[0, pl.ds(c * _CHUNK, _CHUNK), :]
    nearest = near_ref[0, pl.ds(c * _CHUNK, _CHUNK), :]
    mind = jnp.sqrt(jnp.maximum(mind2_ref[0, :, c:c + 1], 1e-12))
    alpha = _BASE_ALPHA * (2.0 - mind / denom)
    r = q + alpha * (nearest - q)
    out_ref[0, pl.ds(c * _CHUNK, _CHUNK), :] = r
    return r


def _k2_body(pred_ref, partial_ref, near_ref, mind2in_ref,
             refined_ref, idx_ref, mind2_ref):
    m = partial_ref.shape[1]
    n_chunks = pred_ref.shape[1] // _CHUNK
    b = pl.program_id(0)
    part = partial_ref[0]
    k2 = jnp.sum(part * part, axis=1)[None, :]
    iota_m = lax.broadcasted_iota(jnp.int32, (_CHUNK, m), 1).astype(jnp.float32)
    denom = jnp.sqrt(jnp.maximum(jnp.max(mind2in_ref[0]), 1e-12)) + 1e-6
    for c in range(n_chunks):
        _update_chunk(pred_ref, near_ref, mind2in_ref, refined_ref, c, denom)
    for c in range(n_chunks):
        mind2, idx = _dist_chunk(refined_ref, c, part, k2, iota_m, m)
        idx_ref[0, 0, pl.ds(c * _CHUNK, _CHUNK)] = (
            idx.astype(jnp.int32) + b * m).reshape(_CHUNK)
        mind2_ref[0, :, c:c + 1] = mind2


def _k3_body(refined_ref, near_ref, mind2in_ref, out_ref):
    n_chunks = refined_ref.shape[1] // _CHUNK
    denom = jnp.sqrt(jnp.maximum(jnp.max(mind2in_ref[0]), 1e-12)) + 1e-6
    for c in range(n_chunks):
        _update_chunk(refined_ref, near_ref, mind2in_ref, out_ref, c, denom)


_ROWW = 128


def _make_sc_gather(total_rows):
    info = plsc.get_sparse_core_info()
    nw = info.num_cores * info.num_subcores
    rows_per_w = total_rows // nw
    idx_rows = rows_per_w // _ROWW
    mesh = plsc.VectorSubcoreMesh(core_axis_name="c", subcore_axis_name="s")

    @functools.partial(
        pl.kernel, mesh=mesh,
        out_type=jax.ShapeDtypeStruct((total_rows, _ROWW), jnp.float32),
        scratch_types=[
            pltpu.VMEM((idx_rows, _ROWW), jnp.int32),
            pltpu.VMEM((rows_per_w, _ROWW), jnp.float32),
            pltpu.SemaphoreType.DMA,
        ],
    )
    def gather_k(table_hbm, idx_hbm, out_hbm, idx_v, rows_v, sem):
        wid = lax.axis_index("s") * info.num_cores + lax.axis_index("c")
        pltpu.sync_copy(idx_hbm.at[pl.ds(wid * idx_rows, idx_rows)], idx_v)
        copies = [
            pltpu.async_copy(
                table_hbm.at[idx_v.at[j]],
                rows_v.at[pl.ds(j * _ROWW, _ROWW)], sem)
            for j in range(idx_rows)
        ]
        for cp in copies:
            cp.wait()
        pltpu.sync_copy(rows_v, out_hbm.at[pl.ds(wid * rows_per_w, rows_per_w)])

    return gather_k


@jax.jit
def kernel(pred, partial):
    b, n, d = pred.shape
    _, m, _ = partial.shape
    n_chunks = n // _CHUNK
    grid_b = (b,)
    spec_nd = pl.BlockSpec((1, n, d), lambda i: (i, 0, 0))
    spec_md = pl.BlockSpec((1, m, d), lambda i: (i, 0, 0))
    spec_scr = pl.BlockSpec((1, _CHUNK, n_chunks), lambda i: (i, 0, 0))
    spec_idx = pl.BlockSpec((1, 1, n), lambda i: (i, 0, 0))
    cp = pltpu.CompilerParams(dimension_semantics=("arbitrary",))

    idx1, mind2_1 = pl.pallas_call(
        _k1_body, grid=grid_b,
        in_specs=[spec_nd, spec_md],
        out_specs=[spec_idx, spec_scr],
        out_shape=[
            jax.ShapeDtypeStruct((b, 1, n), jnp.int32),
            jax.ShapeDtypeStruct((b, _CHUNK, n_chunks), jnp.float32),
        ],
        compiler_params=cp,
    )(pred, partial)

    table = jnp.pad(partial.reshape(b * m, d), ((0, 0), (0, _ROWW - d)))
    gather = _make_sc_gather(b * n)
    idx2d_1 = idx1.reshape(b * n // _ROWW, _ROWW)
    near1 = gather(table, idx2d_1)[:, :d].reshape(b, n, d)

    refined1, idx2, mind2_2 = pl.pallas_call(
        _k2_body, grid=grid_b,
        in_specs=[spec_nd, spec_md, spec_nd, spec_scr],
        out_specs=[spec_nd, spec_idx, spec_scr],
        out_shape=[
            jax.ShapeDtypeStruct((b, n, d), jnp.float32),
            jax.ShapeDtypeStruct((b, 1, n), jnp.int32),
            jax.ShapeDtypeStruct((b, _CHUNK, n_chunks), jnp.float32),
        ],
        compiler_params=cp,
    )(pred, partial, near1, mind2_1)

    idx2d_2 = idx2.reshape(b * n // _ROWW, _ROWW)
    near2 = gather(table, idx2d_2)[:, :d].reshape(b, n, d)

    out = pl.pallas_call(
        _k3_body, grid=grid_b,
        in_specs=[spec_nd, spec_nd, spec_scr],
        out_specs=spec_nd,
        out_shape=jax.ShapeDtypeStruct((b, n, d), jnp.float32),
        compiler_params=cp,
    )(refined1, near2, mind2_2)
    return out


# speedup vs baseline: 1.4208x; 1.4208x over previous
import functools

import jax
import jax.numpy as jnp
from jax import lax
from jax.experimental import pallas as pl
from jax.experimental.pallas import tpu as pltpu

_BASE_ALPHA = 0.05
_NUM_ITER = 2
_CHUNK = 512


def _refine_body(pred_ref, partial_ref, out_ref, mind2_ref, g_ref):
    n = pred_ref.shape[1]
    m = partial_ref.shape[1]
    d = pred_ref.shape[2]
    n_chunks = n // _CHUNK

    part = partial_ref[0]
    part1_bf = jnp.concatenate(
        [part, jnp.ones((m, 1), jnp.float32)], axis=1
    ).astype(jnp.bfloat16)
    k2 = jnp.sum(part * part, axis=1)[None, :]

    for it in range(_NUM_ITER):
        src_ref = pred_ref if it == 0 else out_ref

        def pass1(c, running_max):
            q = src_ref[0, pl.ds(c * _CHUNK, _CHUNK), :]
            qk = lax.dot_general(-2.0 * q, part, (((1,), (1,)), ((), ())),
                                 preferred_element_type=jnp.float32)
            s = qk + k2
            mn = jnp.min(s, axis=1, keepdims=True)
            mask = jnp.where(s <= mn, 1.0, 0.0).astype(jnp.bfloat16)
            g_ref[c] = lax.dot_general(mask, part1_bf,
                                       (((1,), (0,)), ((), ())),
                                       preferred_element_type=jnp.float32)
            q2 = jnp.sum(q * q, axis=1, keepdims=True)
            mind2_ref[c] = q2 + mn
            return jnp.maximum(running_max, jnp.max(q2 + mn))

        max_d2 = jnp.float32(-jnp.inf)
        for c in range(n_chunks):
            max_d2 = pass1(c, max_d2)
        denom = jnp.sqrt(jnp.maximum(max_d2, 1e-12)) + 1e-6

        def pass2(c):
            g = g_ref[c]
            nearest = g[:, :d] / g[:, d:]
            mind = jnp.sqrt(jnp.maximum(mind2_ref[c], 1e-12))
            alpha = _BASE_ALPHA * (2.0 - mind / denom)
            q = src_ref[0, pl.ds(c * _CHUNK, _CHUNK), :]
            out_ref[0, pl.ds(c * _CHUNK, _CHUNK), :] = (
                q + alpha * (nearest - q))

        for c in range(n_chunks):
            pass2(c)


@jax.jit
def kernel(pred, partial):
    b, n, d = pred.shape
    _, m, _ = partial.shape
    n_chunks = n // _CHUNK
    return pl.pallas_call(
        _refine_body,
        grid=(b,),
        in_specs=[
            pl.BlockSpec((1, n, d), lambda i: (i, 0, 0)),
            pl.BlockSpec((1, m, d), lambda i: (i, 0, 0)),
        ],
        out_specs=pl.BlockSpec((1, n, d), lambda i: (i, 0, 0)),
        out_shape=jax.ShapeDtypeStruct((b, n, d), jnp.float32),
        scratch_shapes=[
            pltpu.VMEM((n_chunks, _CHUNK, 1), jnp.float32),
            pltpu.VMEM((n_chunks, _CHUNK, d + 1), jnp.float32),
        ],
        compiler_params=pltpu.CompilerParams(
            dimension_semantics=("arbitrary",),
        ),
    )(pred, partial)


# speedup vs baseline: 1.4929x; 1.0508x over previous
import functools

import jax
import jax.numpy as jnp
from jax import lax
from jax.experimental import pallas as pl
from jax.experimental.pallas import tpu as pltpu

_BASE_ALPHA = 0.05
_NUM_ITER = 2
_CHUNK = 1024


def _refine_body(pred_ref, partial_ref, out_ref, mind2_ref, mask_ref):
    n = pred_ref.shape[1]
    m = partial_ref.shape[1]
    d = pred_ref.shape[2]
    n_chunks = n // _CHUNK

    part = partial_ref[0]
    part1_bf = jnp.concatenate(
        [part, jnp.ones((m, 1), jnp.float32)], axis=1
    ).astype(jnp.bfloat16)
    k2 = jnp.sum(part * part, axis=1)[None, :]

    for it in range(_NUM_ITER):
        src_ref = pred_ref if it == 0 else out_ref

        def pass1(c, running_max):
            q = src_ref[0, pl.ds(c * _CHUNK, _CHUNK), :]
            qk = lax.dot_general(-2.0 * q, part, (((1,), (1,)), ((), ())),
                                 preferred_element_type=jnp.float32)
            s = qk + k2
            mn = jnp.min(s, axis=1, keepdims=True)
            mask_ref[c] = jnp.where(s <= mn, 1.0, 0.0
                                    ).astype(jnp.bfloat16)
            q2 = jnp.sum(q * q, axis=1, keepdims=True)
            mind2_ref[c] = q2 + mn
            return jnp.maximum(running_max, jnp.max(q2 + mn))

        max_d2 = jnp.float32(-jnp.inf)
        for c in range(n_chunks):
            max_d2 = pass1(c, max_d2)
        denom = jnp.sqrt(jnp.maximum(max_d2, 1e-12)) + 1e-6

        def pass2(c):
            g = lax.dot_general(mask_ref[c], part1_bf,
                                (((1,), (0,)), ((), ())),
                                preferred_element_type=jnp.float32)
            nearest = g[:, :d] / g[:, d:]
            mind = jnp.sqrt(jnp.maximum(mind2_ref[c], 1e-12))
            alpha = _BASE_ALPHA * (2.0 - mind / denom)
            q = src_ref[0, pl.ds(c * _CHUNK, _CHUNK), :]
            out_ref[0, pl.ds(c * _CHUNK, _CHUNK), :] = (
                q + alpha * (nearest - q))

        for c in range(n_chunks):
            pass2(c)


@jax.jit
def kernel(pred, partial):
    b, n, d = pred.shape
    _, m, _ = partial.shape
    n_chunks = n // _CHUNK
    return pl.pallas_call(
        _refine_body,
        grid=(b,),
        in_specs=[
            pl.BlockSpec((1, n, d), lambda i: (i, 0, 0)),
            pl.BlockSpec((1, m, d), lambda i: (i, 0, 0)),
        ],
        out_specs=pl.BlockSpec((1, n, d), lambda i: (i, 0, 0)),
        out_shape=jax.ShapeDtypeStruct((b, n, d), jnp.float32),
        scratch_shapes=[
            pltpu.VMEM((n_chunks, _CHUNK, 1), jnp.float32),
            pltpu.VMEM((n_chunks, _CHUNK, m), jnp.bfloat16),
        ],
        compiler_params=pltpu.CompilerParams(
            dimension_semantics=("arbitrary",),
        ),
    )(pred, partial)


# speedup vs baseline: 1.4938x; 1.0006x over previous
import functools

import jax
import jax.numpy as jnp
from jax import lax
from jax.experimental import pallas as pl
from jax.experimental.pallas import tpu as pltpu

_BASE_ALPHA = 0.05
_NUM_ITER = 2
_CHUNK = 512


def _refine_body(pred_ref, partial_ref, out_ref, mind2_ref, mask_ref):
    n = pred_ref.shape[1]
    m = partial_ref.shape[1]
    d = pred_ref.shape[2]
    n_chunks = n // _CHUNK

    part = partial_ref[0]
    part1_bf = jnp.concatenate(
        [part, jnp.ones((m, 1), jnp.float32)], axis=1
    ).astype(jnp.bfloat16)
    k2 = jnp.sum(part * part, axis=1)[None, :]

    for it in range(_NUM_ITER):
        src_ref = pred_ref if it == 0 else out_ref

        def pass1(c, running_max):
            q = src_ref[0, pl.ds(c * _CHUNK, _CHUNK), :]
            qk = lax.dot_general(-2.0 * q, part, (((1,), (1,)), ((), ())),
                                 preferred_element_type=jnp.float32)
            s = qk + k2
            mn = jnp.min(s, axis=1, keepdims=True)
            mask_ref[c] = jnp.where(s <= mn, 1.0, 0.0
                                    ).astype(jnp.bfloat16)
            q2 = jnp.sum(q * q, axis=1, keepdims=True)
            mind2_ref[c] = q2 + mn
            return jnp.maximum(running_max, jnp.max(q2 + mn))

        max_d2 = jnp.float32(-jnp.inf)
        for c in range(n_chunks):
            max_d2 = pass1(c, max_d2)
        denom = jnp.sqrt(jnp.maximum(max_d2, 1e-12)) + 1e-6

        def pass2(c):
            g = lax.dot_general(mask_ref[c], part1_bf,
                                (((1,), (0,)), ((), ())),
                                preferred_element_type=jnp.float32)
            nearest = g[:, :d] / g[:, d:]
            mind = jnp.sqrt(jnp.maximum(mind2_ref[c], 1e-12))
            alpha = _BASE_ALPHA * (2.0 - mind / denom)
            q = src_ref[0, pl.ds(c * _CHUNK, _CHUNK), :]
            out_ref[0, pl.ds(c * _CHUNK, _CHUNK), :] = (
                q + alpha * (nearest - q))

        for c in range(n_chunks):
            pass2(c)


@jax.jit
def kernel(pred, partial):
    b, n, d = pred.shape
    _, m, _ = partial.shape
    n_chunks = n // _CHUNK
    return pl.pallas_call(
        _refine_body,
        grid=(b,),
        in_specs=[
            pl.BlockSpec((1, n, d), lambda i: (i, 0, 0)),
            pl.BlockSpec((1, m, d), lambda i: (i, 0, 0)),
        ],
        out_specs=pl.BlockSpec((1, n, d), lambda i: (i, 0, 0)),
        out_shape=jax.ShapeDtypeStruct((b, n, d), jnp.float32),
        scratch_shapes=[
            pltpu.VMEM((n_chunks, _CHUNK, 1), jnp.float32),
            pltpu.VMEM((n_chunks, _CHUNK, m), jnp.bfloat16),
        ],
        compiler_params=pltpu.CompilerParams(
            dimension_semantics=("arbitrary",),
        ),
    )(pred, partial)
